# baseline (speedup 1.0000x reference)
import functools

import jax
import jax.numpy as jnp
from jax import lax
from jax.experimental import pallas as pl
from jax.experimental.pallas import tpu as pltpu, tpu_sc as plsc

BATCH = 16384
EMBED_K = 32
TROW = 128
RPB = TROW // EMBED_K

_info = plsc.get_sparse_core_info()
_NC, _NS, _L = _info.num_cores, _info.num_subcores, _info.num_lanes
_NW = _NC * _NS
_BPW = BATCH // _NW
_CHUNK = 128
_NCHUNK = _BPW // _CHUNK
_GPC = _CHUNK // _L


def _sc_body(uidx_hbm, iidx_hbm, user_hbm, item_hbm, w1_hbm, w0_hbm,
             y1_hbm, y0_hbm,
             idx_u, idx_i, gidx, uw, iw, w1_v, w0_v, y1_v, y0_v, sem):
    wid = lax.axis_index("s") * _NC + lax.axis_index("c")
    base = wid * _BPW

    pltpu.sync_copy(uidx_hbm.at[pl.ds(wid * _NCHUNK, _NCHUNK)], idx_u)
    pltpu.sync_copy(iidx_hbm.at[pl.ds(wid * _NCHUNK, _NCHUNK)], idx_i)
    pltpu.sync_copy(w1_hbm, w1_v)
    pltpu.sync_copy(w0_hbm, w0_v)

    iota = lax.broadcasted_iota(jnp.int32, (_L,), 0)

    def start_chunk(j):
        for t in range(_CHUNK // _L):
            sl = pl.ds(t * _L, _L)
            gidx[0, sl] = lax.shift_right_logical(idx_u[j, sl], RPB // 2)
            gidx[1, sl] = lax.shift_right_logical(idx_i[j, sl], RPB // 2)
        b = j % 2
        return (
            pltpu.async_copy(user_hbm.at[gidx.at[0]], uw.at[b], sem),
            pltpu.async_copy(item_hbm.at[gidx.at[1]], iw.at[b], sem),
        )

    w1_regs = [w1_v[pl.ds(t * _L, _L)] for t in range(2 * EMBED_K // _L)]
    w0_regs = [w0_v[pl.ds(t * _L, _L)] for t in range(2 * EMBED_K // _L)]

    def _w(regs, k):
        return regs[k // _L][k % _L]

    inflight = start_chunk(0)
    for j in range(_NCHUNK):
        for c in inflight:
            c.wait()
        if j + 1 < _NCHUNK:
            inflight = start_chunk(j + 1)
        b = j % 2

        def group(g, carry, j=j, b=b):
            rows = g * _L + iota
            qu = (idx_u[j, pl.ds(g * _L, _L)] & (RPB - 1)) * EMBED_K
            qi = (idx_i[j, pl.ds(g * _L, _L)] & (RPB - 1)) * EMBED_K
            acc1 = jnp.zeros((_L,), jnp.float32)
            acc0 = jnp.zeros((_L,), jnp.float32)
            for k in range(EMBED_K):
                uv = plsc.load_gather(uw.at[b], [rows, qu + k])
                iv = plsc.load_gather(iw.at[b], [rows, qi + k])
                acc1 = acc1 + uv * _w(w1_regs, k) + iv * _w(w1_regs, EMBED_K + k)
                acc0 = acc0 + uv * _w(w0_regs, k) + iv * _w(w0_regs, EMBED_K + k)
            off = (j * _GPC + g) * _L
            y1_v[pl.ds(off, _L)] = acc1
            y0_v[pl.ds(off, _L)] = acc0
            return carry

        lax.fori_loop(0, _GPC, group, 0, unroll=False)

    pltpu.sync_copy(y1_v, y1_hbm.at[pl.ds(base, _BPW)])
    pltpu.sync_copy(y0_v, y0_hbm.at[pl.ds(base, _BPW)])


@jax.jit
def _sc_call(uidx, iidx, user_table, item_table, w1, w0):
    mesh = plsc.VectorSubcoreMesh(core_axis_name="c", subcore_axis_name="s")
    f = functools.partial(
        pl.kernel,
        mesh=mesh,
        compiler_params=pltpu.CompilerParams(needs_layout_passes=False),
        out_type=(
            jax.ShapeDtypeStruct((BATCH,), jnp.float32),
            jax.ShapeDtypeStruct((BATCH,), jnp.float32),
        ),
        scratch_types=[
            pltpu.VMEM((_NCHUNK, _CHUNK), jnp.int32),
            pltpu.VMEM((_NCHUNK, _CHUNK), jnp.int32),
            pltpu.VMEM((2, _CHUNK), jnp.int32),
            pltpu.VMEM((2, _CHUNK, TROW), jnp.float32),
            pltpu.VMEM((2, _CHUNK, TROW), jnp.float32),
            pltpu.VMEM((2 * EMBED_K,), jnp.float32),
            pltpu.VMEM((2 * EMBED_K,), jnp.float32),
            pltpu.VMEM((_BPW,), jnp.float32),
            pltpu.VMEM((_BPW,), jnp.float32),
            pltpu.SemaphoreType.DMA,
        ],
    )(_sc_body)
    return f(uidx, iidx, user_table, item_table, w1, w0)


def kernel(x, user_table, item_table, W1, W0):
    x = x.astype(jnp.int32)
    uidx = x[:, 0].reshape(_NW * _NCHUNK, _CHUNK)
    iidx = x[:, 1].reshape(_NW * _NCHUNK, _CHUNK)
    ut = user_table.reshape(-1, TROW)
    it = item_table.reshape(-1, TROW)
    w1 = W1.reshape(2 * EMBED_K)
    w0 = W0.reshape(2 * EMBED_K)
    y1, y0 = _sc_call(uidx, iidx, ut, it, w1, w0)
    return (y1.reshape(BATCH, 1), y0.reshape(BATCH, 1))


# speedup vs baseline: 2.5655x; 2.5655x over previous
import functools

import jax
import jax.numpy as jnp
from jax import lax
from jax.experimental import pallas as pl
from jax.experimental.pallas import tpu as pltpu, tpu_sc as plsc

BATCH = 16384
EMBED_K = 32
NROWS = 1000000
BN = 2048
GRID1 = (NROWS + BN - 1) // BN
PADN = GRID1 * BN
TROW = 128

_info = plsc.get_sparse_core_info()
_NC, _NS, _L = _info.num_cores, _info.num_subcores, _info.num_lanes
_NW = _NC * _NS
_BPW = BATCH // _NW
_CHUNK = 128
_NCHUNK = _BPW // _CHUNK
_GPC = _CHUNK // _L


def _tc_body(wu_ref, wi_ref, ut_ref, it_ref, u1_ref, u0_ref, i1_ref, i0_ref):
    ru = lax.dot_general(wu_ref[...], ut_ref[...], (((1,), (0,)), ((), ())),
                         preferred_element_type=jnp.float32)
    ri = lax.dot_general(wi_ref[...], it_ref[...], (((1,), (0,)), ((), ())),
                         preferred_element_type=jnp.float32)
    u1_ref[...] = ru[0]
    u0_ref[...] = ru[1]
    i1_ref[...] = ri[0]
    i0_ref[...] = ri[1]


def _tc_call(wu, wi, ut_t, it_t):
    out1d = jax.ShapeDtypeStruct((PADN,), jnp.float32)
    return pl.pallas_call(
        _tc_body,
        grid=(GRID1,),
        in_specs=[
            pl.BlockSpec((2, EMBED_K), lambda i: (0, 0)),
            pl.BlockSpec((2, EMBED_K), lambda i: (0, 0)),
            pl.BlockSpec((EMBED_K, BN), lambda i: (0, i)),
            pl.BlockSpec((EMBED_K, BN), lambda i: (0, i)),
        ],
        out_specs=[
            pl.BlockSpec((BN,), lambda i: (i,)),
            pl.BlockSpec((BN,), lambda i: (i,)),
            pl.BlockSpec((BN,), lambda i: (i,)),
            pl.BlockSpec((BN,), lambda i: (i,)),
        ],
        out_shape=[out1d, out1d, out1d, out1d],
    )(wu, wi, ut_t, it_t)


def _sc_body(uidx_hbm, iidx_hbm, u1_hbm, u0_hbm, i1_hbm, i0_hbm,
             y1_hbm, y0_hbm,
             idx_u, idx_i, gidx, bu1, bu0, bi1, bi0, y1_v, y0_v, sem):
    wid = lax.axis_index("s") * _NC + lax.axis_index("c")
    base = wid * _BPW

    pltpu.sync_copy(uidx_hbm.at[pl.ds(wid * _NCHUNK, _NCHUNK)], idx_u)
    pltpu.sync_copy(iidx_hbm.at[pl.ds(wid * _NCHUNK, _NCHUNK)], idx_i)

    iota = lax.broadcasted_iota(jnp.int32, (_L,), 0)

    for j in range(_NCHUNK):
        for t in range(_CHUNK // _L):
            sl = pl.ds(t * _L, _L)
            gidx[0, sl] = lax.shift_right_logical(idx_u[j, sl], 7)
            gidx[1, sl] = lax.shift_right_logical(idx_i[j, sl], 7)
        copies = (
            pltpu.async_copy(u1_hbm.at[gidx.at[0]], bu1, sem),
            pltpu.async_copy(u0_hbm.at[gidx.at[0]], bu0, sem),
            pltpu.async_copy(i1_hbm.at[gidx.at[1]], bi1, sem),
            pltpu.async_copy(i0_hbm.at[gidx.at[1]], bi0, sem),
        )
        for c in copies:
            c.wait()

        def group(g, carry, j=j):
            rows = g * _L + iota
            qu = idx_u[j, pl.ds(g * _L, _L)] & (TROW - 1)
            qi = idx_i[j, pl.ds(g * _L, _L)] & (TROW - 1)
            y1 = (plsc.load_gather(bu1, [rows, qu])
                  + plsc.load_gather(bi1, [rows, qi]))
            y0 = (plsc.load_gather(bu0, [rows, qu])
                  + plsc.load_gather(bi0, [rows, qi]))
            off = (j * _GPC + g) * _L
            y1_v[pl.ds(off, _L)] = y1
            y0_v[pl.ds(off, _L)] = y0
            return carry

        lax.fori_loop(0, _GPC, group, 0, unroll=False)

    pltpu.sync_copy(y1_v, y1_hbm.at[pl.ds(base, _BPW)])
    pltpu.sync_copy(y0_v, y0_hbm.at[pl.ds(base, _BPW)])


def _sc_call(uidx, iidx, u1, u0, i1, i0):
    mesh = plsc.VectorSubcoreMesh(core_axis_name="c", subcore_axis_name="s")
    f = functools.partial(
        pl.kernel,
        mesh=mesh,
        compiler_params=pltpu.CompilerParams(needs_layout_passes=False),
        out_type=(
            jax.ShapeDtypeStruct((BATCH,), jnp.float32),
            jax.ShapeDtypeStruct((BATCH,), jnp.float32),
        ),
        scratch_types=[
            pltpu.VMEM((_NCHUNK, _CHUNK), jnp.int32),
            pltpu.VMEM((_NCHUNK, _CHUNK), jnp.int32),
            pltpu.VMEM((2, _CHUNK), jnp.int32),
            pltpu.VMEM((_CHUNK, TROW), jnp.float32),
            pltpu.VMEM((_CHUNK, TROW), jnp.float32),
            pltpu.VMEM((_CHUNK, TROW), jnp.float32),
            pltpu.VMEM((_CHUNK, TROW), jnp.float32),
            pltpu.VMEM((_BPW,), jnp.float32),
            pltpu.VMEM((_BPW,), jnp.float32),
            pltpu.SemaphoreType.DMA,
        ],
    )(_sc_body)
    return f(uidx, iidx, u1, u0, i1, i0)


@jax.jit
def _run(x, user_table, item_table, W1, W0):
    uidx = x[:, 0].reshape(_NW * _NCHUNK, _CHUNK)
    iidx = x[:, 1].reshape(_NW * _NCHUNK, _CHUNK)
    wu = jnp.concatenate([W1[:, :EMBED_K], W0[:, :EMBED_K]], axis=0)
    wi = jnp.concatenate([W1[:, EMBED_K:], W0[:, EMBED_K:]], axis=0)
    u1, u0, i1, i0 = _tc_call(wu, wi, user_table.T, item_table.T)
    rb = PADN // TROW
    y1, y0 = _sc_call(uidx, iidx, u1.reshape(rb, TROW), u0.reshape(rb, TROW),
                      i1.reshape(rb, TROW), i0.reshape(rb, TROW))
    return (y1.reshape(BATCH, 1), y0.reshape(BATCH, 1))


def kernel(x, user_table, item_table, W1, W0):
    return _run(x.astype(jnp.int32), user_table, item_table, W1, W0)


# speedup vs baseline: 5.5796x; 2.1749x over previous
import functools

import jax
import jax.numpy as jnp
from jax import lax
from jax.experimental import pallas as pl
from jax.experimental.pallas import tpu as pltpu, tpu_sc as plsc

BATCH = 16384
EMBED_K = 32
NROWS = 1000000
BN = 8192
GRID1 = (NROWS + BN - 1) // BN
PADN = GRID1 * BN
TROW = 128

_info = plsc.get_sparse_core_info()
_NC, _NS, _L = _info.num_cores, _info.num_subcores, _info.num_lanes
_NW = _NC * _NS
_BPW = BATCH // _NW
_CHUNK = 128
_NCHUNK = _BPW // _CHUNK
_GPC = _CHUNK // _L


def _tc_body(wu_ref, wi_ref, ut_ref, it_ref, u1_ref, u0_ref, i1_ref, i0_ref):
    ru = lax.dot_general(wu_ref[...], ut_ref[...], (((1,), (0,)), ((), ())),
                         preferred_element_type=jnp.float32)
    ri = lax.dot_general(wi_ref[...], it_ref[...], (((1,), (0,)), ((), ())),
                         preferred_element_type=jnp.float32)
    u1_ref[...] = ru[0]
    u0_ref[...] = ru[1]
    i1_ref[...] = ri[0]
    i0_ref[...] = ri[1]


def _tc_call(wu, wi, ut_t, it_t):
    out1d = jax.ShapeDtypeStruct((PADN,), jnp.float32)
    return pl.pallas_call(
        _tc_body,
        grid=(GRID1,),
        in_specs=[
            pl.BlockSpec((2, EMBED_K), lambda i: (0, 0)),
            pl.BlockSpec((2, EMBED_K), lambda i: (0, 0)),
            pl.BlockSpec((EMBED_K, BN), lambda i: (0, i)),
            pl.BlockSpec((EMBED_K, BN), lambda i: (0, i)),
        ],
        out_specs=[
            pl.BlockSpec((BN,), lambda i: (i,)),
            pl.BlockSpec((BN,), lambda i: (i,)),
            pl.BlockSpec((BN,), lambda i: (i,)),
            pl.BlockSpec((BN,), lambda i: (i,)),
        ],
        out_shape=[out1d, out1d, out1d, out1d],
    )(wu, wi, ut_t, it_t)


def _sc_body(uidx_hbm, iidx_hbm, u1_hbm, u0_hbm, i1_hbm, i0_hbm,
             y1_hbm, y0_hbm,
             idx_u, idx_i, gidx, bu1, bu0, bi1, bi0, y1_v, y0_v, sem):
    wid = lax.axis_index("s") * _NC + lax.axis_index("c")
    base = wid * _BPW

    pltpu.sync_copy(uidx_hbm.at[pl.ds(wid * _NCHUNK, _NCHUNK)], idx_u)
    pltpu.sync_copy(iidx_hbm.at[pl.ds(wid * _NCHUNK, _NCHUNK)], idx_i)

    iota = lax.broadcasted_iota(jnp.int32, (_L,), 0)

    for j in range(_NCHUNK):
        for t in range(_CHUNK // _L):
            sl = pl.ds(t * _L, _L)
            gidx[0, sl] = lax.shift_right_logical(idx_u[j, sl], 7)
            gidx[1, sl] = lax.shift_right_logical(idx_i[j, sl], 7)
        copies = (
            pltpu.async_copy(u1_hbm.at[gidx.at[0]], bu1, sem),
            pltpu.async_copy(u0_hbm.at[gidx.at[0]], bu0, sem),
            pltpu.async_copy(i1_hbm.at[gidx.at[1]], bi1, sem),
            pltpu.async_copy(i0_hbm.at[gidx.at[1]], bi0, sem),
        )
        for c in copies:
            c.wait()

        def group(g, carry, j=j):
            rows = g * _L + iota
            qu = idx_u[j, pl.ds(g * _L, _L)] & (TROW - 1)
            qi = idx_i[j, pl.ds(g * _L, _L)] & (TROW - 1)
            y1 = (plsc.load_gather(bu1, [rows, qu])
                  + plsc.load_gather(bi1, [rows, qi]))
            y0 = (plsc.load_gather(bu0, [rows, qu])
                  + plsc.load_gather(bi0, [rows, qi]))
            off = (j * _GPC + g) * _L
            y1_v[pl.ds(off, _L)] = y1
            y0_v[pl.ds(off, _L)] = y0
            return carry

        lax.fori_loop(0, _GPC, group, 0, unroll=False)

    pltpu.sync_copy(y1_v, y1_hbm.at[pl.ds(base, _BPW)])
    pltpu.sync_copy(y0_v, y0_hbm.at[pl.ds(base, _BPW)])


def _sc_call(uidx, iidx, u1, u0, i1, i0):
    mesh = plsc.VectorSubcoreMesh(core_axis_name="c", subcore_axis_name="s")
    f = functools.partial(
        pl.kernel,
        mesh=mesh,
        compiler_params=pltpu.CompilerParams(needs_layout_passes=False),
        out_type=(
            jax.ShapeDtypeStruct((BATCH,), jnp.float32),
            jax.ShapeDtypeStruct((BATCH,), jnp.float32),
        ),
        scratch_types=[
            pltpu.VMEM((_NCHUNK, _CHUNK), jnp.int32),
            pltpu.VMEM((_NCHUNK, _CHUNK), jnp.int32),
            pltpu.VMEM((2, _CHUNK), jnp.int32),
            pltpu.VMEM((_CHUNK, TROW), jnp.float32),
            pltpu.VMEM((_CHUNK, TROW), jnp.float32),
            pltpu.VMEM((_CHUNK, TROW), jnp.float32),
            pltpu.VMEM((_CHUNK, TROW), jnp.float32),
            pltpu.VMEM((_BPW,), jnp.float32),
            pltpu.VMEM((_BPW,), jnp.float32),
            pltpu.SemaphoreType.DMA,
        ],
    )(_sc_body)
    return f(uidx, iidx, u1, u0, i1, i0)


@jax.jit
def _run(x, user_table, item_table, W1, W0):
    uidx = x[:, 0].reshape(_NW * _NCHUNK, _CHUNK)
    iidx = x[:, 1].reshape(_NW * _NCHUNK, _CHUNK)
    wu = jnp.concatenate([W1[:, :EMBED_K], W0[:, :EMBED_K]], axis=0)
    wi = jnp.concatenate([W1[:, EMBED_K:], W0[:, EMBED_K:]], axis=0)
    u1, u0, i1, i0 = _tc_call(wu, wi, user_table.T, item_table.T)
    rb = PADN // TROW
    y1, y0 = _sc_call(uidx, iidx, u1.reshape(rb, TROW), u0.reshape(rb, TROW),
                      i1.reshape(rb, TROW), i0.reshape(rb, TROW))
    return (y1.reshape(BATCH, 1), y0.reshape(BATCH, 1))


def kernel(x, user_table, item_table, W1, W0):
    return _run(x.astype(jnp.int32), user_table, item_table, W1, W0)


# speedup vs baseline: 7.1463x; 1.2808x over previous
import functools

import jax
import jax.numpy as jnp
from jax import lax
from jax.experimental import pallas as pl
from jax.experimental.pallas import tpu as pltpu, tpu_sc as plsc

BATCH = 16384
EMBED_K = 32
NROWS = 1000000
BN = 16384
GRID1 = (NROWS + BN - 1) // BN
PADN = GRID1 * BN
TROW = 128

_info = plsc.get_sparse_core_info()
_NC, _NS, _L = _info.num_cores, _info.num_subcores, _info.num_lanes
_NW = _NC * _NS
_BPW = BATCH // _NW
_CHUNK = 128
_NCHUNK = _BPW // _CHUNK
_GPC = _CHUNK // _L


def _tc_body(wu_ref, wi_ref, ut_ref, it_ref, u1_ref, u0_ref, i1_ref, i0_ref):
    ru = lax.dot_general(wu_ref[...], ut_ref[...], (((1,), (0,)), ((), ())),
                         preferred_element_type=jnp.float32)
    ri = lax.dot_general(wi_ref[...], it_ref[...], (((1,), (0,)), ((), ())),
                         preferred_element_type=jnp.float32)
    u1_ref[...] = ru[0]
    u0_ref[...] = ru[1]
    i1_ref[...] = ri[0]
    i0_ref[...] = ri[1]


def _tc_call(wu, wi, ut_t, it_t):
    out1d = jax.ShapeDtypeStruct((PADN,), jnp.float32)
    return pl.pallas_call(
        _tc_body,
        grid=(GRID1,),
        in_specs=[
            pl.BlockSpec((2, EMBED_K), lambda i: (0, 0)),
            pl.BlockSpec((2, EMBED_K), lambda i: (0, 0)),
            pl.BlockSpec((EMBED_K, BN), lambda i: (0, i)),
            pl.BlockSpec((EMBED_K, BN), lambda i: (0, i)),
        ],
        out_specs=[
            pl.BlockSpec((BN,), lambda i: (i,)),
            pl.BlockSpec((BN,), lambda i: (i,)),
            pl.BlockSpec((BN,), lambda i: (i,)),
            pl.BlockSpec((BN,), lambda i: (i,)),
        ],
        out_shape=[out1d, out1d, out1d, out1d],
    )(wu, wi, ut_t, it_t)


def _sc_body(uidx_hbm, iidx_hbm, u1_hbm, u0_hbm, i1_hbm, i0_hbm,
             y1_hbm, y0_hbm,
             idx_u, idx_i, gidx, bu1, bu0, bi1, bi0, y1_v, y0_v, sem):
    wid = lax.axis_index("s") * _NC + lax.axis_index("c")
    base = wid * _BPW

    pltpu.sync_copy(uidx_hbm.at[pl.ds(wid * _NCHUNK, _NCHUNK)], idx_u)
    pltpu.sync_copy(iidx_hbm.at[pl.ds(wid * _NCHUNK, _NCHUNK)], idx_i)

    iota = lax.broadcasted_iota(jnp.int32, (_L,), 0)

    for j in range(_NCHUNK):
        for t in range(_CHUNK // _L):
            sl = pl.ds(t * _L, _L)
            gidx[0, sl] = lax.shift_right_logical(idx_u[j, sl], 7)
            gidx[1, sl] = lax.shift_right_logical(idx_i[j, sl], 7)
        copies = (
            pltpu.async_copy(u1_hbm.at[gidx.at[0]], bu1, sem),
            pltpu.async_copy(u0_hbm.at[gidx.at[0]], bu0, sem),
            pltpu.async_copy(i1_hbm.at[gidx.at[1]], bi1, sem),
            pltpu.async_copy(i0_hbm.at[gidx.at[1]], bi0, sem),
        )
        for c in copies:
            c.wait()

        def group(g, carry, j=j):
            rows = g * _L + iota
            qu = idx_u[j, pl.ds(g * _L, _L)] & (TROW - 1)
            qi = idx_i[j, pl.ds(g * _L, _L)] & (TROW - 1)
            y1 = (plsc.load_gather(bu1, [rows, qu])
                  + plsc.load_gather(bi1, [rows, qi]))
            y0 = (plsc.load_gather(bu0, [rows, qu])
                  + plsc.load_gather(bi0, [rows, qi]))
            off = (j * _GPC + g) * _L
            y1_v[pl.ds(off, _L)] = y1
            y0_v[pl.ds(off, _L)] = y0
            return carry

        lax.fori_loop(0, _GPC, group, 0, unroll=False)

    pltpu.sync_copy(y1_v, y1_hbm.at[pl.ds(base, _BPW)])
    pltpu.sync_copy(y0_v, y0_hbm.at[pl.ds(base, _BPW)])


def _sc_call(uidx, iidx, u1, u0, i1, i0):
    mesh = plsc.VectorSubcoreMesh(core_axis_name="c", subcore_axis_name="s")
    f = functools.partial(
        pl.kernel,
        mesh=mesh,
        compiler_params=pltpu.CompilerParams(needs_layout_passes=False),
        out_type=(
            jax.ShapeDtypeStruct((BATCH,), jnp.float32),
            jax.ShapeDtypeStruct((BATCH,), jnp.float32),
        ),
        scratch_types=[
            pltpu.VMEM((_NCHUNK, _CHUNK), jnp.int32),
            pltpu.VMEM((_NCHUNK, _CHUNK), jnp.int32),
            pltpu.VMEM((2, _CHUNK), jnp.int32),
            pltpu.VMEM((_CHUNK, TROW), jnp.float32),
            pltpu.VMEM((_CHUNK, TROW), jnp.float32),
            pltpu.VMEM((_CHUNK, TROW), jnp.float32),
            pltpu.VMEM((_CHUNK, TROW), jnp.float32),
            pltpu.VMEM((_BPW,), jnp.float32),
            pltpu.VMEM((_BPW,), jnp.float32),
            pltpu.SemaphoreType.DMA,
        ],
    )(_sc_body)
    return f(uidx, iidx, u1, u0, i1, i0)


@jax.jit
def _run(x, user_table, item_table, W1, W0):
    uidx = x[:, 0].reshape(_NW * _NCHUNK, _CHUNK)
    iidx = x[:, 1].reshape(_NW * _NCHUNK, _CHUNK)
    wu = jnp.concatenate([W1[:, :EMBED_K], W0[:, :EMBED_K]], axis=0)
    wi = jnp.concatenate([W1[:, EMBED_K:], W0[:, EMBED_K:]], axis=0)
    u1, u0, i1, i0 = _tc_call(wu, wi, user_table.T, item_table.T)
    rb = PADN // TROW
    y1, y0 = _sc_call(uidx, iidx, u1.reshape(rb, TROW), u0.reshape(rb, TROW),
                      i1.reshape(rb, TROW), i0.reshape(rb, TROW))
    return (y1.reshape(BATCH, 1), y0.reshape(BATCH, 1))


def kernel(x, user_table, item_table, W1, W0):
    return _run(x.astype(jnp.int32), user_table, item_table, W1, W0)


# speedup vs baseline: 7.7131x; 1.0793x over previous
import functools

import jax
import jax.numpy as jnp
from jax import lax
from jax.experimental import pallas as pl
from jax.experimental.pallas import tpu as pltpu, tpu_sc as plsc

BATCH = 16384
EMBED_K = 32
NROWS = 1000000
BN = 32768
GRID1 = (NROWS + BN - 1) // BN
PADN = GRID1 * BN
TROW = 128

_info = plsc.get_sparse_core_info()
_NC, _NS, _L = _info.num_cores, _info.num_subcores, _info.num_lanes
_NW = _NC * _NS
_BPW = BATCH // _NW
_CHUNK = 128
_NCHUNK = _BPW // _CHUNK
_GPC = _CHUNK // _L


def _tc_body(wu_ref, wi_ref, ut_ref, it_ref, u1_ref, u0_ref, i1_ref, i0_ref):
    ru = lax.dot_general(wu_ref[...], ut_ref[...], (((1,), (0,)), ((), ())),
                         preferred_element_type=jnp.float32)
    ri = lax.dot_general(wi_ref[...], it_ref[...], (((1,), (0,)), ((), ())),
                         preferred_element_type=jnp.float32)
    u1_ref[...] = ru[0]
    u0_ref[...] = ru[1]
    i1_ref[...] = ri[0]
    i0_ref[...] = ri[1]


def _tc_call(wu, wi, ut_t, it_t):
    out1d = jax.ShapeDtypeStruct((PADN,), jnp.float32)
    return pl.pallas_call(
        _tc_body,
        grid=(GRID1,),
        in_specs=[
            pl.BlockSpec((2, EMBED_K), lambda i: (0, 0)),
            pl.BlockSpec((2, EMBED_K), lambda i: (0, 0)),
            pl.BlockSpec((EMBED_K, BN), lambda i: (0, i)),
            pl.BlockSpec((EMBED_K, BN), lambda i: (0, i)),
        ],
        out_specs=[
            pl.BlockSpec((BN,), lambda i: (i,)),
            pl.BlockSpec((BN,), lambda i: (i,)),
            pl.BlockSpec((BN,), lambda i: (i,)),
            pl.BlockSpec((BN,), lambda i: (i,)),
        ],
        out_shape=[out1d, out1d, out1d, out1d],
    )(wu, wi, ut_t, it_t)


def _sc_body(uidx_hbm, iidx_hbm, u1_hbm, u0_hbm, i1_hbm, i0_hbm,
             y1_hbm, y0_hbm,
             idx_u, idx_i, gidx, bu1, bu0, bi1, bi0, y1_v, y0_v, sem):
    wid = lax.axis_index("s") * _NC + lax.axis_index("c")
    base = wid * _BPW

    pltpu.sync_copy(uidx_hbm.at[pl.ds(wid * _NCHUNK, _NCHUNK)], idx_u)
    pltpu.sync_copy(iidx_hbm.at[pl.ds(wid * _NCHUNK, _NCHUNK)], idx_i)

    iota = lax.broadcasted_iota(jnp.int32, (_L,), 0)

    for j in range(_NCHUNK):
        for t in range(_CHUNK // _L):
            sl = pl.ds(t * _L, _L)
            gidx[0, sl] = lax.shift_right_logical(idx_u[j, sl], 7)
            gidx[1, sl] = lax.shift_right_logical(idx_i[j, sl], 7)
        copies = (
            pltpu.async_copy(u1_hbm.at[gidx.at[0]], bu1, sem),
            pltpu.async_copy(u0_hbm.at[gidx.at[0]], bu0, sem),
            pltpu.async_copy(i1_hbm.at[gidx.at[1]], bi1, sem),
            pltpu.async_copy(i0_hbm.at[gidx.at[1]], bi0, sem),
        )
        for c in copies:
            c.wait()

        def group(g, carry, j=j):
            rows = g * _L + iota
            qu = idx_u[j, pl.ds(g * _L, _L)] & (TROW - 1)
            qi = idx_i[j, pl.ds(g * _L, _L)] & (TROW - 1)
            y1 = (plsc.load_gather(bu1, [rows, qu])
                  + plsc.load_gather(bi1, [rows, qi]))
            y0 = (plsc.load_gather(bu0, [rows, qu])
                  + plsc.load_gather(bi0, [rows, qi]))
            off = (j * _GPC + g) * _L
            y1_v[pl.ds(off, _L)] = y1
            y0_v[pl.ds(off, _L)] = y0
            return carry

        lax.fori_loop(0, _GPC, group, 0, unroll=False)

    pltpu.sync_copy(y1_v, y1_hbm.at[pl.ds(base, _BPW)])
    pltpu.sync_copy(y0_v, y0_hbm.at[pl.ds(base, _BPW)])


def _sc_call(uidx, iidx, u1, u0, i1, i0):
    mesh = plsc.VectorSubcoreMesh(core_axis_name="c", subcore_axis_name="s")
    f = functools.partial(
        pl.kernel,
        mesh=mesh,
        compiler_params=pltpu.CompilerParams(needs_layout_passes=False),
        out_type=(
            jax.ShapeDtypeStruct((BATCH,), jnp.float32),
            jax.ShapeDtypeStruct((BATCH,), jnp.float32),
        ),
        scratch_types=[
            pltpu.VMEM((_NCHUNK, _CHUNK), jnp.int32),
            pltpu.VMEM((_NCHUNK, _CHUNK), jnp.int32),
            pltpu.VMEM((2, _CHUNK), jnp.int32),
            pltpu.VMEM((_CHUNK, TROW), jnp.float32),
            pltpu.VMEM((_CHUNK, TROW), jnp.float32),
            pltpu.VMEM((_CHUNK, TROW), jnp.float32),
            pltpu.VMEM((_CHUNK, TROW), jnp.float32),
            pltpu.VMEM((_BPW,), jnp.float32),
            pltpu.VMEM((_BPW,), jnp.float32),
            pltpu.SemaphoreType.DMA,
        ],
    )(_sc_body)
    return f(uidx, iidx, u1, u0, i1, i0)


@jax.jit
def _run(x, user_table, item_table, W1, W0):
    uidx = x[:, 0].reshape(_NW * _NCHUNK, _CHUNK)
    iidx = x[:, 1].reshape(_NW * _NCHUNK, _CHUNK)
    wu = jnp.concatenate([W1[:, :EMBED_K], W0[:, :EMBED_K]], axis=0)
    wi = jnp.concatenate([W1[:, EMBED_K:], W0[:, EMBED_K:]], axis=0)
    u1, u0, i1, i0 = _tc_call(wu, wi, user_table.T, item_table.T)
    rb = PADN // TROW
    y1, y0 = _sc_call(uidx, iidx, u1.reshape(rb, TROW), u0.reshape(rb, TROW),
                      i1.reshape(rb, TROW), i0.reshape(rb, TROW))
    return (y1.reshape(BATCH, 1), y0.reshape(BATCH, 1))


def kernel(x, user_table, item_table, W1, W0):
    return _run(x.astype(jnp.int32), user_table, item_table, W1, W0)
